# baseline (speedup 1.0000x reference)
import functools

import jax
import jax.numpy as jnp
from jax import lax
from jax.experimental import pallas as pl
from jax.experimental.pallas import tpu as pltpu
from jax.experimental.pallas import tpu_sc as plsc

_LB = 128
_PAIR = 2
_NBUF = 2


@functools.cache
def _make_gather(nb, nl, num_emb, d):
    info = plsc.get_sparse_core_info()
    nc, ns = info.num_cores, info.num_subcores
    nw = nc * ns
    nblk = nb // _LB
    blk_w = nblk // nw
    pairs = blk_w // _PAIR
    groups = nl * pairs
    rows_n = _PAIR * _LB
    dh_n = d // 8
    line = 8 * _LB
    stripe = -(-num_emb // ns)
    assert nblk % nw == 0 and blk_w % _PAIR == 0 and d % 8 == 0
    assert groups % _NBUF == 0
    mesh = plsc.VectorSubcoreMesh(core_axis_name="c", subcore_axis_name="s")

    @functools.partial(
        pl.kernel,
        mesh=mesh,
        out_type=jax.ShapeDtypeStruct((nl * dh_n * nblk, line), jnp.float32),
        scratch_types=[
            pltpu.VMEM_SHARED((num_emb, d), jnp.float32),
            pltpu.VMEM((_PAIR, _LB), jnp.int32),
            pltpu.VMEM((_PAIR, _LB), jnp.int32),
            pltpu.VMEM((rows_n, d), jnp.float32),
            pltpu.VMEM((rows_n, d), jnp.float32),
            pltpu.VMEM((dh_n, _PAIR, line), jnp.float32),
            pltpu.VMEM((dh_n, _PAIR, line), jnp.float32),
            pltpu.SemaphoreType.DMA,
            pltpu.SemaphoreType.DMA,
            pltpu.SemaphoreType.DMA,
            pltpu.SemaphoreType.DMA,
            pltpu.SemaphoreType.DMA,
        ],
        compiler_params=pltpu.CompilerParams(
            use_tc_tiling_on_sc=False, needs_layout_passes=False
        ),
    )
    def gather_kernel(idx_hbm, table_hbm, out_hbm, table_sh, idx0, idx1,
                      rows0, rows1, tr0, tr1,
                      sem_i0, sem_i1, sem_g, sem_o0, sem_o1):
        idx_v = (idx0, idx1)
        rows_v = (rows0, rows1)
        tr_v = (tr0, tr1)
        sem_i = (sem_i0, sem_i1)
        sem_o = (sem_o0, sem_o1)
        sid = lax.axis_index("s")
        wid = sid * nc + lax.axis_index("c")
        blk0 = wid * blk_w

        start = jnp.minimum(sid * stripe, num_emb - stripe)
        pltpu.sync_copy(
            table_hbm.at[pl.ds(start, stripe)], table_sh.at[pl.ds(start, stripe)]
        )
        plsc.subcore_barrier()

        iota16 = lax.iota(jnp.int32, 16)
        cols = [jnp.full((16,), dd, jnp.int32) for dd in range(d)]

        def idx_src(g):
            gc = jnp.minimum(g, groups - 1)
            l, p = gc // pairs, gc % pairs
            return idx_hbm.at[pl.ds(l * nblk + blk0 + p * _PAIR, _PAIR)]

        def do_group(g, b, wait_out):
            if wait_out:
                for dh in range(dh_n):
                    pltpu.make_async_copy(
                        tr_v[b].at[dh], out_hbm.at[pl.ds(0, _PAIR)], sem_o[b]
                    ).wait()
            pltpu.make_async_copy(idx_src(0), idx_v[b], sem_i[b]).wait()
            copies = [
                pltpu.async_copy(
                    table_sh.at[idx_v[b].at[j]],
                    rows_v[b].at[pl.ds(j * _LB, _LB)],
                    sem_g,
                )
                for j in range(_PAIR)
            ]
            for c in copies:
                c.wait()
            pltpu.async_copy(idx_src(g + _NBUF), idx_v[b], sem_i[b])
            for j in range(_PAIR):
                for c16 in range(0, _LB, 16):
                    rv = iota16 + (j * _LB + c16)
                    for dd in range(d):
                        vals = plsc.load_gather(rows_v[b], [rv, cols[dd]])
                        tr_v[b][dd // 8, j,
                                pl.ds((dd % 8) * _LB + c16, 16)] = vals
            l, p = g // pairs, g % pairs
            for dh in range(dh_n):
                row = (l * dh_n + dh) * nblk + blk0 + p * _PAIR
                pltpu.async_copy(
                    tr_v[b].at[dh], out_hbm.at[pl.ds(row, _PAIR)], sem_o[b]
                )

        for b in range(_NBUF):
            pltpu.async_copy(idx_src(jnp.int32(b)), idx_v[b], sem_i[b])
        for b in range(_NBUF):
            do_group(jnp.int32(b), b, wait_out=False)

        def pair_body(p, carry):
            for b in range(_NBUF):
                do_group(p * _NBUF + b, b, wait_out=True)
            return carry

        lax.fori_loop(1, groups // _NBUF, pair_body, 0)

        for b in range(_NBUF):
            pltpu.make_async_copy(idx_src(0), idx_v[b], sem_i[b]).wait()
            for dh in range(dh_n):
                pltpu.make_async_copy(
                    tr_v[b].at[dh], out_hbm.at[pl.ds(0, _PAIR)], sem_o[b]
                ).wait()

    return gather_kernel


def kernel(sent_list, table):
    nb, nl = sent_list.shape
    d = table.shape[1]
    idx2 = sent_list.T.reshape(nl * (nb // _LB), _LB).astype(jnp.int32)
    out2d = _make_gather(nb, nl, table.shape[0], d)(
        idx2, table.astype(jnp.float32)
    )
    out5 = out2d.reshape(nl, d // 8, nb // _LB, 8, _LB)
    return out5.transpose(2, 4, 0, 1, 3).reshape(nb, nl, d)


# speedup vs baseline: 1.5294x; 1.5294x over previous
import functools

import jax
import jax.numpy as jnp
from jax import lax
from jax.experimental import pallas as pl
from jax.experimental.pallas import tpu as pltpu
from jax.experimental.pallas import tpu_sc as plsc

_BLK = 128
_K = 5
_NBUF = 2


@functools.cache
def _make_gather(n_total, num_emb, d):
    info = plsc.get_sparse_core_info()
    nc, ns = info.num_cores, info.num_subcores
    nw = nc * ns
    blocks_total = n_total // _BLK
    blocks_w = blocks_total // nw
    groups = blocks_w // _K
    assert groups % _NBUF == 0
    rows_per_group = _K * _BLK
    stripe = -(-num_emb // ns)
    mesh = plsc.VectorSubcoreMesh(core_axis_name="c", subcore_axis_name="s")

    @functools.partial(
        pl.kernel,
        mesh=mesh,
        out_type=jax.ShapeDtypeStruct((n_total, d), jnp.float32),
        scratch_types=[
            pltpu.VMEM_SHARED((num_emb, d), jnp.float32),
            pltpu.VMEM((_K, _BLK), jnp.int32),
            pltpu.VMEM((_K, _BLK), jnp.int32),
            pltpu.VMEM((rows_per_group, d), jnp.float32),
            pltpu.VMEM((rows_per_group, d), jnp.float32),
            pltpu.SemaphoreType.DMA,
            pltpu.SemaphoreType.DMA,
            pltpu.SemaphoreType.DMA,
            pltpu.SemaphoreType.DMA,
            pltpu.SemaphoreType.DMA,
        ],
        compiler_params=pltpu.CompilerParams(use_tc_tiling_on_sc=False),
    )
    def gather_kernel(idx_hbm, table_hbm, out_hbm, table_sh, idx0, idx1,
                      rows0, rows1, sem_i0, sem_i1, sem_g, sem_o0, sem_o1):
        idx_v = (idx0, idx1)
        rows_v = (rows0, rows1)
        sem_i = (sem_i0, sem_i1)
        sem_o = (sem_o0, sem_o1)
        sid = lax.axis_index("s")
        wid = sid * nc + lax.axis_index("c")
        blk0 = wid * blocks_w

        start = jnp.minimum(sid * stripe, num_emb - stripe)
        pltpu.sync_copy(
            table_hbm.at[pl.ds(start, stripe)], table_sh.at[pl.ds(start, stripe)]
        )
        plsc.subcore_barrier()

        def idx_src(g):
            return idx_hbm.at[pl.ds(blk0 + g * _K, _K)]

        def do_group(g, b, wait_out):
            if wait_out:
                pltpu.make_async_copy(
                    rows_v[b], out_hbm.at[pl.ds(0, rows_per_group)], sem_o[b]
                ).wait()
            pltpu.make_async_copy(idx_src(0), idx_v[b], sem_i[b]).wait()
            copies = [
                pltpu.async_copy(
                    table_sh.at[idx_v[b].at[j]],
                    rows_v[b].at[pl.ds(j * _BLK, _BLK)],
                    sem_g,
                )
                for j in range(_K)
            ]
            for c in copies:
                c.wait()
            gn = jnp.minimum(g + _NBUF, groups - 1)
            pltpu.async_copy(idx_src(gn), idx_v[b], sem_i[b])
            pltpu.async_copy(
                rows_v[b],
                out_hbm.at[pl.ds((blk0 + g * _K) * _BLK, rows_per_group)],
                sem_o[b],
            )

        for b in range(_NBUF):
            pltpu.async_copy(idx_src(b), idx_v[b], sem_i[b])
        for b in range(_NBUF):
            do_group(jnp.int32(b), b, wait_out=False)

        def pair(p, carry):
            for b in range(_NBUF):
                do_group(p * _NBUF + b, b, wait_out=True)
            return carry

        lax.fori_loop(1, groups // _NBUF, pair, 0)

        for b in range(_NBUF):
            pltpu.make_async_copy(idx_src(0), idx_v[b], sem_i[b]).wait()
            pltpu.make_async_copy(
                rows_v[b], out_hbm.at[pl.ds(0, rows_per_group)], sem_o[b]
            ).wait()

    return gather_kernel


def kernel(sent_list, table):
    b, l = sent_list.shape
    n_total = b * l
    d = table.shape[1]
    idx2d = sent_list.reshape(n_total // _BLK, _BLK).astype(jnp.int32)
    out = _make_gather(n_total, table.shape[0], d)(idx2d, table.astype(jnp.float32))
    return out.reshape(b, l, d)


# speedup vs baseline: 1.5835x; 1.0354x over previous
import functools

import jax
import jax.numpy as jnp
from jax import lax
from jax.experimental import pallas as pl
from jax.experimental.pallas import tpu as pltpu
from jax.experimental.pallas import tpu_sc as plsc

_LB = 128
_PAIR = 2
_NBUF = 2


@functools.cache
def _make_gather(nb, nl, num_emb, d):
    info = plsc.get_sparse_core_info()
    nc, ns = info.num_cores, info.num_subcores
    nw = nc * ns
    nblk = nb // _LB
    blk_w = nblk // nw
    pairs = blk_w // _PAIR
    groups = nl * pairs
    rows_n = _PAIR * _LB
    dh_n = d // 8
    line = 8 * _LB
    stripe = -(-num_emb // ns)
    assert nblk % nw == 0 and blk_w % _PAIR == 0 and d % 8 == 0
    assert groups % _NBUF == 0
    mesh = plsc.VectorSubcoreMesh(core_axis_name="c", subcore_axis_name="s")

    @functools.partial(
        pl.kernel,
        mesh=mesh,
        out_type=jax.ShapeDtypeStruct((nl * dh_n * nblk, line), jnp.float32),
        scratch_types=[
            pltpu.VMEM_SHARED((num_emb, d), jnp.float32),
            pltpu.VMEM((_PAIR, _LB), jnp.int32),
            pltpu.VMEM((_PAIR, _LB), jnp.int32),
            pltpu.VMEM((rows_n, d), jnp.float32),
            pltpu.VMEM((rows_n, d), jnp.float32),
            pltpu.VMEM((dh_n, _PAIR, line), jnp.float32),
            pltpu.VMEM((dh_n, _PAIR, line), jnp.float32),
            pltpu.SemaphoreType.DMA,
            pltpu.SemaphoreType.DMA,
            pltpu.SemaphoreType.DMA,
            pltpu.SemaphoreType.DMA,
            pltpu.SemaphoreType.DMA,
        ],
        compiler_params=pltpu.CompilerParams(
            use_tc_tiling_on_sc=False, needs_layout_passes=False
        ),
    )
    def gather_kernel(idx_hbm, table_hbm, out_hbm, table_sh, idx0, idx1,
                      rows0, rows1, tr0, tr1,
                      sem_i0, sem_i1, sem_g, sem_o0, sem_o1):
        idx_v = (idx0, idx1)
        rows_v = (rows0, rows1)
        tr_v = (tr0, tr1)
        sem_i = (sem_i0, sem_i1)
        sem_o = (sem_o0, sem_o1)
        sid = lax.axis_index("s")
        wid = sid * nc + lax.axis_index("c")
        blk0 = wid * blk_w

        start = jnp.minimum(sid * stripe, num_emb - stripe)
        pltpu.sync_copy(
            table_hbm.at[pl.ds(start, stripe)], table_sh.at[pl.ds(start, stripe)]
        )
        plsc.subcore_barrier()

        iota16 = lax.iota(jnp.int32, 16)
        cols = [jnp.full((16,), dd, jnp.int32) for dd in range(d)]

        def idx_src(g):
            gc = jnp.minimum(g, groups - 1)
            l, p = gc // pairs, gc % pairs
            return idx_hbm.at[pl.ds(l * nblk + blk0 + p * _PAIR, _PAIR)]

        def do_group(g, b, wait_out):
            if wait_out:
                for dh in range(dh_n):
                    pltpu.make_async_copy(
                        tr_v[b].at[dh], out_hbm.at[pl.ds(0, _PAIR)], sem_o[b]
                    ).wait()
            pltpu.make_async_copy(idx_src(0), idx_v[b], sem_i[b]).wait()
            copies = [
                pltpu.async_copy(
                    table_sh.at[idx_v[b].at[j]],
                    rows_v[b].at[pl.ds(j * _LB, _LB)],
                    sem_g,
                )
                for j in range(_PAIR)
            ]
            for c in copies:
                c.wait()
            pltpu.async_copy(idx_src(g + _NBUF), idx_v[b], sem_i[b])
            @plsc.parallel_loop(0, rows_n // 16, unroll=2)
            def _chunk(jc):
                rv = iota16 + jc * 16
                j = jc // (_LB // 16)
                c16 = (jc % (_LB // 16)) * 16
                for dd in range(d):
                    vals = plsc.load_gather(rows_v[b], [rv, cols[dd]])
                    tr_v[b][dd // 8, j,
                            pl.ds((dd % 8) * _LB + c16, 16)] = vals
            l, p = g // pairs, g % pairs
            for dh in range(dh_n):
                row = (l * dh_n + dh) * nblk + blk0 + p * _PAIR
                pltpu.async_copy(
                    tr_v[b].at[dh], out_hbm.at[pl.ds(row, _PAIR)], sem_o[b]
                )

        for b in range(_NBUF):
            pltpu.async_copy(idx_src(jnp.int32(b)), idx_v[b], sem_i[b])
        for b in range(_NBUF):
            do_group(jnp.int32(b), b, wait_out=False)

        def pair_body(p, carry):
            for b in range(_NBUF):
                do_group(p * _NBUF + b, b, wait_out=True)
            return carry

        lax.fori_loop(1, groups // _NBUF, pair_body, 0)

        for b in range(_NBUF):
            pltpu.make_async_copy(idx_src(0), idx_v[b], sem_i[b]).wait()
            for dh in range(dh_n):
                pltpu.make_async_copy(
                    tr_v[b].at[dh], out_hbm.at[pl.ds(0, _PAIR)], sem_o[b]
                ).wait()

    return gather_kernel


def kernel(sent_list, table):
    nb, nl = sent_list.shape
    d = table.shape[1]
    idx2 = sent_list.T.reshape(nl * (nb // _LB), _LB).astype(jnp.int32)
    out2d = _make_gather(nb, nl, table.shape[0], d)(
        idx2, table.astype(jnp.float32)
    )
    out5 = out2d.reshape(nl, d // 8, nb // _LB, 8, _LB)
    return out5.transpose(2, 4, 0, 1, 3).reshape(nb, nl, d)
